# Initial kernel scaffold; baseline (speedup 1.0000x reference)
import jax
import jax.numpy as jnp
from jax.experimental import pallas as pl


def kernel(xyz, points, W0, b0, g0, be0, W1, b1, g1, be1, W2, b2, g2, be2):
    raise NotImplementedError("write your pallas kernel here")


# speedup vs baseline: 13.2741x; 13.2741x over previous
import functools

import jax
import jax.numpy as jnp
from jax import lax
from jax.experimental import pallas as pl
from jax.experimental.pallas import tpu as pltpu
from jax.experimental.pallas import tpu_sc as plsc

B = 8
N = 4096
S = 512
K = 32
CP = 125
CIN = 128
RADIUS2 = 0.2 ** 2
EPS = 1e-5
ROWS = B * S * K
BLK = 2048


def _fps_body(x_ref, y_ref, z_ref, out_ref):
    X = x_ref[...]
    Y = y_ref[...]
    Z = z_ref[...]
    n_iota = lax.broadcasted_iota(jnp.int32, (B, N), 1)

    def step(i, carry):
        dist, far = carry
        out_ref[i] = far
        onehot = n_iota == far
        cx = jnp.sum(jnp.where(onehot, X, 0.0), axis=1, keepdims=True)
        cy = jnp.sum(jnp.where(onehot, Y, 0.0), axis=1, keepdims=True)
        cz = jnp.sum(jnp.where(onehot, Z, 0.0), axis=1, keepdims=True)
        d = (X - cx) ** 2 + (Y - cy) ** 2 + (Z - cz) ** 2
        dist = jnp.minimum(dist, d)
        m = jnp.max(dist, axis=1, keepdims=True)
        far = jnp.max(jnp.where(dist == m, n_iota, -1), axis=1, keepdims=True)
        return dist, far

    init = (jnp.full((B, N), 1e10, jnp.float32),
            jnp.zeros((B, 1), jnp.int32))
    lax.fori_loop(0, S, step, init)


def _fps(xyz):
    x = xyz[:, :, 0]
    y = xyz[:, :, 1]
    z = xyz[:, :, 2]
    out = pl.pallas_call(
        _fps_body,
        out_shape=jax.ShapeDtypeStruct((S, B, 1), jnp.int32),
    )(x, y, z)
    return out[:, :, 0].T


def _bf16_round(v):
    bits = plsc.bitcast(v, jnp.int32)
    lsb = jnp.bitwise_and(lax.shift_right_logical(bits, 16), 1)
    r = jnp.bitwise_and(bits + 0x7FFF + lsb, jnp.int32(-65536))
    return plsc.bitcast(r, jnp.float32)


def _sc_group(xf, yf, zf, fps_flat, table):
    mesh = plsc.VectorSubcoreMesh(core_axis_name="c", subcore_axis_name="s",
                                  num_cores=2, num_subcores=16)

    @functools.partial(
        pl.kernel,
        out_type=(jax.ShapeDtypeStruct((ROWS, CIN), jnp.float32),
                  jax.ShapeDtypeStruct((B * S * 3,), jnp.float32)),
        mesh=mesh,
        compiler_params=pltpu.CompilerParams(needs_layout_passes=False),
        scratch_types=[
            pltpu.VMEM((N,), jnp.float32),
            pltpu.VMEM((N,), jnp.float32),
            pltpu.VMEM((N,), jnp.float32),
            pltpu.VMEM((N,), jnp.float32),
            pltpu.VMEM((N,), jnp.float32),
            pltpu.VMEM((N,), jnp.float32),
            pltpu.VMEM((N,), jnp.float32),
            pltpu.VMEM((128,), jnp.int32),
            pltpu.VMEM((128,), jnp.float32),
            pltpu.VMEM((128,), jnp.float32),
            pltpu.VMEM((128,), jnp.float32),
            pltpu.VMEM((384,), jnp.float32),
            pltpu.VMEM((48,), jnp.int32),
            pltpu.VMEM((128,), jnp.int32),
            pltpu.VMEM((128,), jnp.int32),
            pltpu.VMEM((128, CIN), jnp.float32),
            pltpu.VMEM((128, CIN), jnp.float32),
            pltpu.SemaphoreType.DMA,
        ],
    )
    def grouping(x_hbm, y_hbm, z_hbm, fps_hbm, t_hbm, g_hbm, nxz_hbm,
                 xv, yv, zv, xr, yr, zr, p2v, sidx, cxb, cyb, czb, nxs, idxb,
                 glA, glB, rbA, rbB, sem):
        cid = lax.axis_index("c")
        sid = lax.axis_index("s")
        w = sid * 2 + cid
        b = w // 4
        s0 = (w % 4) * 128
        lane = lax.iota(jnp.int32, 16)

        pltpu.sync_copy(x_hbm.at[pl.ds(b * N, N)], xv)
        pltpu.sync_copy(y_hbm.at[pl.ds(b * N, N)], yv)
        pltpu.sync_copy(z_hbm.at[pl.ds(b * N, N)], zv)
        pltpu.sync_copy(fps_hbm.at[pl.ds(b * S + s0, 128)], sidx)

        def prep(t, _):
            bse = t * 16
            xs = xv[pl.ds(bse, 16)]
            ys = yv[pl.ds(bse, 16)]
            zs = zv[pl.ds(bse, 16)]
            xr[pl.ds(bse, 16)] = _bf16_round(xs)
            yr[pl.ds(bse, 16)] = _bf16_round(ys)
            zr[pl.ds(bse, 16)] = _bf16_round(zs)
            p2v[pl.ds(bse, 16)] = (xs * xs + ys * ys) + zs * zs
            return 0

        lax.fori_loop(0, N // 16, prep, 0)

        for j in range(8):
            iv = sidx[pl.ds(j * 16, 16)]
            gx = plsc.load_gather(xv, [iv])
            gy = plsc.load_gather(yv, [iv])
            gz = plsc.load_gather(zv, [iv])
            cxb[pl.ds(j * 16, 16)] = gx
            cyb[pl.ds(j * 16, 16)] = gy
            czb[pl.ds(j * 16, 16)] = gz
            offs = (j * 16 + lane) * 3
            plsc.store_scatter(nxs, [offs], gx)
            plsc.store_scatter(nxs, [offs + 1], gy)
            plsc.store_scatter(nxs, [offs + 2], gz)
        pltpu.sync_copy(nxs, nxz_hbm.at[pl.ds((b * 512 + s0) * 3, 384)])

        def ball_query(sl, glist, slot):
            sv = jnp.full((16,), sl, jnp.int32)
            cxv = plsc.load_gather(cxb, [sv])
            cyv = plsc.load_gather(cyb, [sv])
            czv = plsc.load_gather(czb, [sv])
            c2 = (cxv * cxv + cyv * cyv) + czv * czv
            crx = _bf16_round(cxv)
            cry = _bf16_round(cyv)
            crz = _bf16_round(czv)

            def cond(c):
                t, cnt, _ = c
                return jnp.logical_and(cnt < 32, t < N // 16)

            def body(c):
                t, cnt, firstv = c
                base = t * 16
                xs = xr[pl.ds(base, 16)]
                ys = yr[pl.ds(base, 16)]
                zs = zr[pl.ds(base, 16)]
                dot = (crx * xs + cry * ys) + crz * zs
                d = ((-2.0 * dot) + c2) + p2v[pl.ds(base, 16)]
                msk = jnp.logical_not(d > RADIUS2)
                plsc.store_compressed(idxb.at[pl.ds(cnt, 16)], base + lane,
                                      mask=msk)
                pc = plsc.all_reduce_population_count(msk)
                ffs = plsc.all_reduce_ffs(msk)
                hit0 = jnp.logical_and(cnt == 0,
                                       jnp.logical_and(ffs >= 0, ffs < 16))
                firstv = jnp.where(hit0, base + ffs, firstv)
                return t + 1, cnt + jnp.max(pc), firstv

            _, cnt, firstv = lax.while_loop(
                cond, body, (jnp.int32(0), jnp.int32(0), sv))
            for g in range(2):
                pos = g * 16 + lane
                cur = idxb[pl.ds(g * 16, 16)]
                sel = jnp.where(pos < cnt, cur, firstv)
                glist[pl.ds(slot * 32 + g * 16, 16)] = sel + b * N

        def grp(grp_i, _):
            sbase = grp_i * 8

            def cents_a(i, _):
                ball_query(sbase + i, glA, i)
                return 0

            def cents_b(i, _):
                ball_query(sbase + 4 + i, glB, i)
                return 0

            lax.fori_loop(0, 4, cents_a, 0)
            lax.fori_loop(0, 4, cents_b, 0)

            cpA = pltpu.async_copy(t_hbm.at[glA], rbA, sem)
            cpB = pltpu.async_copy(t_hbm.at[glB], rbB, sem)
            cpA.wait()
            cpB.wait()

            for half, rb in ((0, rbA), (1, rbB)):
                for j in range(8):
                    rid = j * 16 + lane
                    slv = sbase + half * 4 + lax.shift_right_logical(rid, 5)
                    for ci, cb in enumerate((cxb, cyb, czb)):
                        cv = plsc.load_gather(cb, [slv])
                        col = jnp.full((16,), ci, jnp.int32)
                        val = plsc.load_gather(rb, [rid, col]) - cv
                        plsc.store_scatter(rb, [rid, col], val)

            row0 = (b * 512 + s0 + sbase) * 32
            pltpu.sync_copy(rbA, g_hbm.at[pl.ds(row0, 128)])
            pltpu.sync_copy(rbB, g_hbm.at[pl.ds(row0 + 128, 128)])
            return 0

        lax.fori_loop(0, 16, grp, 0)

    return grouping(xf, yf, zf, fps_flat, table)


def _stats(y):
    ssum = jnp.sum(y.reshape(BLK // 8, 8, y.shape[-1]), axis=0)
    ssq = jnp.sum((y * y).reshape(BLK // 8, 8, y.shape[-1]), axis=0)
    return jnp.concatenate([ssum, ssq], axis=0)


def _norm_from_stats(s_ref, g_ref, be_ref, y):
    s = s_ref[...]
    mean = jnp.sum(s[0:8], axis=0, keepdims=True) / ROWS
    ex2 = jnp.sum(s[8:16], axis=0, keepdims=True) / ROWS
    var = ex2 - mean * mean
    inv = lax.rsqrt(var + EPS)
    z = g_ref[...] * ((y - mean) * inv) + be_ref[...]
    return jnp.maximum(z, 0.0)


def _passA_body(g_ref, w_ref, b_ref, y_ref, s_ref):
    i = pl.program_id(0)
    y = jnp.dot(g_ref[...], w_ref[...],
                preferred_element_type=jnp.float32) + b_ref[...]
    y_ref[...] = y
    st = _stats(y)

    @pl.when(i == 0)
    def _():
        s_ref[...] = st

    @pl.when(i > 0)
    def _():
        s_ref[...] = s_ref[...] + st


def _passB_body(y0_ref, s0_ref, g0_ref, be0_ref, w_ref, b_ref, y_ref, s_ref):
    i = pl.program_id(0)
    x = _norm_from_stats(s0_ref, g0_ref, be0_ref, y0_ref[...])
    y = jnp.dot(x, w_ref[...], preferred_element_type=jnp.float32) + b_ref[...]
    y_ref[...] = y
    st = _stats(y)

    @pl.when(i == 0)
    def _():
        s_ref[...] = st

    @pl.when(i > 0)
    def _():
        s_ref[...] = s_ref[...] + st


def _passC_body(y1_ref, s1_ref, g1_ref, be1_ref, w_ref, b_ref,
                mx_ref, mn_ref, s_ref):
    i = pl.program_id(0)
    x = _norm_from_stats(s1_ref, g1_ref, be1_ref, y1_ref[...])
    y = jnp.dot(x, w_ref[...], preferred_element_type=jnp.float32) + b_ref[...]
    st = _stats(y)
    z = y.reshape(BLK // K, K, y.shape[-1])
    mx_ref[...] = jnp.max(z, axis=1)
    mn_ref[...] = jnp.min(z, axis=1)

    @pl.when(i == 0)
    def _():
        s_ref[...] = st

    @pl.when(i > 0)
    def _():
        s_ref[...] = s_ref[...] + st


def _passD_body(mx_ref, mn_ref, s2_ref, g2_ref, be2_ref, o_ref):
    g2 = g2_ref[...]
    sel = jnp.where(g2 >= 0.0, mx_ref[...], mn_ref[...])
    o_ref[...] = _norm_from_stats(s2_ref, g2_ref, be2_ref, sel)


def _mlp(G, W0, b0, g0, be0, W1, b1, g1, be1, W2, b2, g2, be2):
    f32 = jnp.float32
    grid = (ROWS // BLK,)
    row_spec = lambda c: pl.BlockSpec((BLK, c), lambda i: (i, 0))
    const_spec = lambda r, c: pl.BlockSpec((r, c), lambda i: (0, 0))

    Y0, S0 = pl.pallas_call(
        _passA_body,
        grid=grid,
        in_specs=[row_spec(CIN), const_spec(CIN, 128), const_spec(1, 128)],
        out_specs=[row_spec(128), const_spec(16, 128)],
        out_shape=[jax.ShapeDtypeStruct((ROWS, 128), f32),
                   jax.ShapeDtypeStruct((16, 128), f32)],
    )(G, W0.T, b0[None, :])

    Y1, S1 = pl.pallas_call(
        _passB_body,
        grid=grid,
        in_specs=[row_spec(128), const_spec(16, 128), const_spec(1, 128),
                  const_spec(1, 128), const_spec(128, 128), const_spec(1, 128)],
        out_specs=[row_spec(128), const_spec(16, 128)],
        out_shape=[jax.ShapeDtypeStruct((ROWS, 128), f32),
                   jax.ShapeDtypeStruct((16, 128), f32)],
    )(Y0, S0, g0[None, :], be0[None, :], W1.T, b1[None, :])

    MX, MN, S2 = pl.pallas_call(
        _passC_body,
        grid=grid,
        in_specs=[row_spec(128), const_spec(16, 128), const_spec(1, 128),
                  const_spec(1, 128), const_spec(128, 256), const_spec(1, 256)],
        out_specs=[pl.BlockSpec((BLK // K, 256), lambda i: (i, 0)),
                   pl.BlockSpec((BLK // K, 256), lambda i: (i, 0)),
                   const_spec(16, 256)],
        out_shape=[jax.ShapeDtypeStruct((ROWS // K, 256), f32),
                   jax.ShapeDtypeStruct((ROWS // K, 256), f32),
                   jax.ShapeDtypeStruct((16, 256), f32)],
    )(Y1, S1, g1[None, :], be1[None, :], W2.T, b2[None, :])

    out = pl.pallas_call(
        _passD_body,
        out_shape=jax.ShapeDtypeStruct((ROWS // K, 256), f32),
    )(MX, MN, S2, g2[None, :], be2[None, :])
    return out


def kernel(xyz, points, W0, b0, g0, be0, W1, b1, g1, be1, W2, b2, g2, be2):
    fps_idx = _fps(xyz)
    table = jnp.concatenate([xyz, points], axis=-1)
    table = table.reshape(B * N, CIN)
    G, nxz = _sc_group(xyz[:, :, 0].reshape(-1), xyz[:, :, 1].reshape(-1),
                       xyz[:, :, 2].reshape(-1), fps_idx.reshape(-1), table)
    out = _mlp(G, W0, b0, g0, be0, W1, b1, g1, be1, W2, b2, g2, be2)
    new_xyz = nxz.reshape(B, S, 3)
    new_points = out.reshape(B, S, 256)
    return new_xyz, new_points


# speedup vs baseline: 16.0509x; 1.2092x over previous
import functools

import jax
import jax.numpy as jnp
from jax import lax
from jax.experimental import pallas as pl
from jax.experimental.pallas import tpu as pltpu
from jax.experimental.pallas import tpu_sc as plsc

B = 8
N = 4096
S = 512
K = 32
CP = 125
CIN = 128
RADIUS2 = 0.2 ** 2
EPS = 1e-5
ROWS = B * S * K
BLK = 2048


def _fps_body(x_ref, y_ref, z_ref, out_ref):
    X = x_ref[...]
    Y = y_ref[...]
    Z = z_ref[...]
    n_iota = lax.broadcasted_iota(jnp.int32, (B, N), 1)

    def step(i, carry):
        dist, far = carry
        out_ref[i] = far
        onehot = n_iota == far
        cx = jnp.sum(jnp.where(onehot, X, 0.0), axis=1, keepdims=True)
        cy = jnp.sum(jnp.where(onehot, Y, 0.0), axis=1, keepdims=True)
        cz = jnp.sum(jnp.where(onehot, Z, 0.0), axis=1, keepdims=True)
        d = (X - cx) ** 2 + (Y - cy) ** 2 + (Z - cz) ** 2
        dist = jnp.minimum(dist, d)
        m = jnp.max(dist, axis=1, keepdims=True)
        far = jnp.max(jnp.where(dist == m, n_iota, -1), axis=1, keepdims=True)
        return dist, far

    init = (jnp.full((B, N), 1e10, jnp.float32),
            jnp.zeros((B, 1), jnp.int32))
    lax.fori_loop(0, S, step, init)


def _fps(xyz):
    x = xyz[:, :, 0]
    y = xyz[:, :, 1]
    z = xyz[:, :, 2]
    out = pl.pallas_call(
        _fps_body,
        out_shape=jax.ShapeDtypeStruct((S, B, 1), jnp.int32),
    )(x, y, z)
    return out[:, :, 0].T


def _bf16_round(v):
    bits = plsc.bitcast(v, jnp.int32)
    lsb = jnp.bitwise_and(lax.shift_right_logical(bits, 16), 1)
    r = jnp.bitwise_and(bits + 0x7FFF + lsb, jnp.int32(-65536))
    return plsc.bitcast(r, jnp.float32)


def _sc_group(xf, yf, zf, fps_flat, table):
    mesh = plsc.VectorSubcoreMesh(core_axis_name="c", subcore_axis_name="s",
                                  num_cores=2, num_subcores=16)

    @functools.partial(
        pl.kernel,
        out_type=(jax.ShapeDtypeStruct((ROWS, CIN), jnp.float32),
                  jax.ShapeDtypeStruct((B * S * 3,), jnp.float32)),
        mesh=mesh,
        compiler_params=pltpu.CompilerParams(needs_layout_passes=False),
        scratch_types=[
            pltpu.VMEM((N,), jnp.float32),
            pltpu.VMEM((N,), jnp.float32),
            pltpu.VMEM((N,), jnp.float32),
            pltpu.VMEM((N,), jnp.float32),
            pltpu.VMEM((N,), jnp.float32),
            pltpu.VMEM((N,), jnp.float32),
            pltpu.VMEM((N,), jnp.float32),
            pltpu.VMEM((128,), jnp.int32),
            pltpu.VMEM((128,), jnp.float32),
            pltpu.VMEM((128,), jnp.float32),
            pltpu.VMEM((128,), jnp.float32),
            pltpu.VMEM((384,), jnp.float32),
            pltpu.VMEM((160,), jnp.int32),
            pltpu.VMEM((128,), jnp.int32),
            pltpu.VMEM((128,), jnp.int32),
            pltpu.VMEM((128,), jnp.int32),
            pltpu.VMEM((128,), jnp.int32),
            pltpu.VMEM((128, CIN), jnp.float32),
            pltpu.VMEM((128, CIN), jnp.float32),
            pltpu.VMEM((128, CIN), jnp.float32),
            pltpu.VMEM((128, CIN), jnp.float32),
            pltpu.SemaphoreType.DMA,
            pltpu.SemaphoreType.DMA,
            pltpu.SemaphoreType.DMA,
            pltpu.SemaphoreType.DMA,
        ],
    )
    def grouping(x_hbm, y_hbm, z_hbm, fps_hbm, t_hbm, g_hbm, nxz_hbm,
                 xv, yv, zv, xr, yr, zr, p2v, sidx, cxb, cyb, czb, nxs, idxb,
                 gl0A, gl0B, gl1A, gl1B, rb0A, rb0B, rb1A, rb1B,
                 gsem0, gsem1, osem0, osem1):
        cid = lax.axis_index("c")
        sid = lax.axis_index("s")
        w = sid * 2 + cid
        b = w // 4
        s0 = (w % 4) * 128
        lane = lax.iota(jnp.int32, 16)

        pltpu.sync_copy(x_hbm.at[pl.ds(b * N, N)], xv)
        pltpu.sync_copy(y_hbm.at[pl.ds(b * N, N)], yv)
        pltpu.sync_copy(z_hbm.at[pl.ds(b * N, N)], zv)
        pltpu.sync_copy(fps_hbm.at[pl.ds(b * S + s0, 128)], sidx)

        def prep(t, _):
            bse = t * 16
            xs = xv[pl.ds(bse, 16)]
            ys = yv[pl.ds(bse, 16)]
            zs = zv[pl.ds(bse, 16)]
            xr[pl.ds(bse, 16)] = _bf16_round(xs)
            yr[pl.ds(bse, 16)] = _bf16_round(ys)
            zr[pl.ds(bse, 16)] = _bf16_round(zs)
            p2v[pl.ds(bse, 16)] = (xs * xs + ys * ys) + zs * zs
            return 0

        lax.fori_loop(0, N // 16, prep, 0)

        for j in range(8):
            iv = sidx[pl.ds(j * 16, 16)]
            gx = plsc.load_gather(xv, [iv])
            gy = plsc.load_gather(yv, [iv])
            gz = plsc.load_gather(zv, [iv])
            cxb[pl.ds(j * 16, 16)] = gx
            cyb[pl.ds(j * 16, 16)] = gy
            czb[pl.ds(j * 16, 16)] = gz
            offs = (j * 16 + lane) * 3
            plsc.store_scatter(nxs, [offs], gx)
            plsc.store_scatter(nxs, [offs + 1], gy)
            plsc.store_scatter(nxs, [offs + 2], gz)
        pltpu.sync_copy(nxs, nxz_hbm.at[pl.ds((b * 512 + s0) * 3, 384)])

        def ball_query(sl, glist, slot):
            sv = jnp.full((16,), sl, jnp.int32)
            cxv = plsc.load_gather(cxb, [sv])
            cyv = plsc.load_gather(cyb, [sv])
            czv = plsc.load_gather(czb, [sv])
            c2 = (cxv * cxv + cyv * cyv) + czv * czv
            crx = _bf16_round(cxv)
            cry = _bf16_round(cyv)
            crz = _bf16_round(czv)
            idxb[pl.ds(0, 16)] = jnp.full((16,), N, jnp.int32)

            def cond(c):
                blk, cnt, _ = c
                return jnp.logical_and(cnt < 32, blk < N // 128)

            def body(c):
                blk, cnt, cntv = c
                base0 = blk * 128
                for j in range(8):
                    base = base0 + j * 16
                    xs = xr[pl.ds(base, 16)]
                    ys = yr[pl.ds(base, 16)]
                    zs = zr[pl.ds(base, 16)]
                    dot = (crx * xs + cry * ys) + crz * zs
                    d = ((-2.0 * dot) + c2) + p2v[pl.ds(base, 16)]
                    msk = jnp.logical_not(d > RADIUS2)
                    prefix = plsc.cumsum(jnp.where(msk, 1, 0))
                    plsc.store_scatter(idxb, [cntv + prefix - 1], base + lane,
                                       mask=msk)
                    cntv = cntv + plsc.all_reduce_population_count(msk)
                return blk + 1, jnp.max(cntv), cntv

            zerov = sv - sv
            _, cnt, _ = lax.while_loop(
                cond, body, (jnp.int32(0), jnp.int32(0), zerov))
            firstv = jnp.full((16,), jnp.min(idxb[pl.ds(0, 16)]), jnp.int32)
            for g in range(2):
                pos = g * 16 + lane
                cur = idxb[pl.ds(g * 16, 16)]
                sel = jnp.where(pos < cnt, cur, firstv)
                glist[pl.ds(slot * 32 + g * 16, 16)] = sel + b * N

        def bq_group(gi, glistA, glistB):
            sbase = gi * 8

            def ca(i, _):
                ball_query(sbase + i, glistA, i)
                return 0

            def cb(i, _):
                ball_query(sbase + 4 + i, glistB, i)
                return 0

            lax.fori_loop(0, 4, ca, 0)
            lax.fori_loop(0, 4, cb, 0)

        def process(gi, rbA, rbB, osem):
            sbase = gi * 8
            for half, rb in ((0, rbA), (1, rbB)):
                for j in range(8):
                    rid = j * 16 + lane
                    slv = sbase + half * 4 + lax.shift_right_logical(rid, 5)
                    for ci, cb in enumerate((cxb, cyb, czb)):
                        cv = plsc.load_gather(cb, [slv])
                        col = jnp.full((16,), ci, jnp.int32)
                        val = plsc.load_gather(rb, [rid, col]) - cv
                        plsc.store_scatter(rb, [rid, col], val)
            row0 = (b * 512 + s0 + sbase) * 32
            pltpu.async_copy(rbA, g_hbm.at[pl.ds(row0, 128)], osem)
            pltpu.async_copy(rbB, g_hbm.at[pl.ds(row0 + 128, 128)], osem)

        def fire_gather(glistA, glistB, rbA, rbB, sem):
            pltpu.async_copy(t_hbm.at[glistA], rbA, sem)
            pltpu.async_copy(t_hbm.at[glistB], rbB, sem)

        def drain2_gather(sem):
            pltpu.make_async_copy(t_hbm.at[pl.ds(0, 128)], rb0A, sem).wait()
            pltpu.make_async_copy(t_hbm.at[pl.ds(0, 128)], rb0B, sem).wait()

        def drain2_out(sem):
            pltpu.make_async_copy(rb0A, g_hbm.at[pl.ds(0, 128)], sem).wait()
            pltpu.make_async_copy(rb0B, g_hbm.at[pl.ds(0, 128)], sem).wait()

        bq_group(0, gl0A, gl0B)
        fire_gather(gl0A, gl0B, rb0A, rb0B, gsem0)

        def pipe(g2, _):
            base = g2 * 2

            bq_group(base + 1, gl1A, gl1B)

            @pl.when(g2 > 0)
            def _():
                drain2_out(osem1)
            fire_gather(gl1A, gl1B, rb1A, rb1B, gsem1)
            drain2_gather(gsem0)
            process(base, rb0A, rb0B, osem0)

            @pl.when(base + 2 < 16)
            def _():
                bq_group(base + 2, gl0A, gl0B)
                drain2_out(osem0)
                fire_gather(gl0A, gl0B, rb0A, rb0B, gsem0)
            drain2_gather(gsem1)
            process(base + 1, rb1A, rb1B, osem1)
            return 0

        lax.fori_loop(0, 8, pipe, 0)
        drain2_out(osem0)
        drain2_out(osem1)

    return grouping(xf, yf, zf, fps_flat, table)


def _stats(y):
    ssum = jnp.sum(y.reshape(BLK // 8, 8, y.shape[-1]), axis=0)
    ssq = jnp.sum((y * y).reshape(BLK // 8, 8, y.shape[-1]), axis=0)
    return jnp.concatenate([ssum, ssq], axis=0)


def _norm_from_stats(s_ref, g_ref, be_ref, y):
    s = s_ref[...]
    mean = jnp.sum(s[0:8], axis=0, keepdims=True) / ROWS
    ex2 = jnp.sum(s[8:16], axis=0, keepdims=True) / ROWS
    var = ex2 - mean * mean
    inv = lax.rsqrt(var + EPS)
    z = g_ref[...] * ((y - mean) * inv) + be_ref[...]
    return jnp.maximum(z, 0.0)


def _passA_body(g_ref, w_ref, b_ref, y_ref, s_ref):
    i = pl.program_id(0)
    y = jnp.dot(g_ref[...], w_ref[...],
                preferred_element_type=jnp.float32) + b_ref[...]
    y_ref[...] = y
    st = _stats(y)

    @pl.when(i == 0)
    def _():
        s_ref[...] = st

    @pl.when(i > 0)
    def _():
        s_ref[...] = s_ref[...] + st


def _passB_body(y0_ref, s0_ref, g0_ref, be0_ref, w_ref, b_ref, y_ref, s_ref):
    i = pl.program_id(0)
    x = _norm_from_stats(s0_ref, g0_ref, be0_ref, y0_ref[...])
    y = jnp.dot(x, w_ref[...], preferred_element_type=jnp.float32) + b_ref[...]
    y_ref[...] = y
    st = _stats(y)

    @pl.when(i == 0)
    def _():
        s_ref[...] = st

    @pl.when(i > 0)
    def _():
        s_ref[...] = s_ref[...] + st


def _passC_body(y1_ref, s1_ref, g1_ref, be1_ref, w_ref, b_ref,
                mx_ref, mn_ref, s_ref):
    i = pl.program_id(0)
    x = _norm_from_stats(s1_ref, g1_ref, be1_ref, y1_ref[...])
    y = jnp.dot(x, w_ref[...], preferred_element_type=jnp.float32) + b_ref[...]
    st = _stats(y)
    z = y.reshape(BLK // K, K, y.shape[-1])
    mx_ref[...] = jnp.max(z, axis=1)
    mn_ref[...] = jnp.min(z, axis=1)

    @pl.when(i == 0)
    def _():
        s_ref[...] = st

    @pl.when(i > 0)
    def _():
        s_ref[...] = s_ref[...] + st


def _passD_body(mx_ref, mn_ref, s2_ref, g2_ref, be2_ref, o_ref):
    g2 = g2_ref[...]
    sel = jnp.where(g2 >= 0.0, mx_ref[...], mn_ref[...])
    o_ref[...] = _norm_from_stats(s2_ref, g2_ref, be2_ref, sel)


def _mlp(G, W0, b0, g0, be0, W1, b1, g1, be1, W2, b2, g2, be2):
    f32 = jnp.float32
    grid = (ROWS // BLK,)
    row_spec = lambda c: pl.BlockSpec((BLK, c), lambda i: (i, 0))
    const_spec = lambda r, c: pl.BlockSpec((r, c), lambda i: (0, 0))

    Y0, S0 = pl.pallas_call(
        _passA_body,
        grid=grid,
        in_specs=[row_spec(CIN), const_spec(CIN, 128), const_spec(1, 128)],
        out_specs=[row_spec(128), const_spec(16, 128)],
        out_shape=[jax.ShapeDtypeStruct((ROWS, 128), f32),
                   jax.ShapeDtypeStruct((16, 128), f32)],
    )(G, W0.T, b0[None, :])

    Y1, S1 = pl.pallas_call(
        _passB_body,
        grid=grid,
        in_specs=[row_spec(128), const_spec(16, 128), const_spec(1, 128),
                  const_spec(1, 128), const_spec(128, 128), const_spec(1, 128)],
        out_specs=[row_spec(128), const_spec(16, 128)],
        out_shape=[jax.ShapeDtypeStruct((ROWS, 128), f32),
                   jax.ShapeDtypeStruct((16, 128), f32)],
    )(Y0, S0, g0[None, :], be0[None, :], W1.T, b1[None, :])

    MX, MN, S2 = pl.pallas_call(
        _passC_body,
        grid=grid,
        in_specs=[row_spec(128), const_spec(16, 128), const_spec(1, 128),
                  const_spec(1, 128), const_spec(128, 256), const_spec(1, 256)],
        out_specs=[pl.BlockSpec((BLK // K, 256), lambda i: (i, 0)),
                   pl.BlockSpec((BLK // K, 256), lambda i: (i, 0)),
                   const_spec(16, 256)],
        out_shape=[jax.ShapeDtypeStruct((ROWS // K, 256), f32),
                   jax.ShapeDtypeStruct((ROWS // K, 256), f32),
                   jax.ShapeDtypeStruct((16, 256), f32)],
    )(Y1, S1, g1[None, :], be1[None, :], W2.T, b2[None, :])

    out = pl.pallas_call(
        _passD_body,
        out_shape=jax.ShapeDtypeStruct((ROWS // K, 256), f32),
    )(MX, MN, S2, g2[None, :], be2[None, :])
    return out


def kernel(xyz, points, W0, b0, g0, be0, W1, b1, g1, be1, W2, b2, g2, be2):
    fps_idx = _fps(xyz)
    table = jnp.concatenate([xyz, points], axis=-1)
    table = table.reshape(B * N, CIN)
    G, nxz = _sc_group(xyz[:, :, 0].reshape(-1), xyz[:, :, 1].reshape(-1),
                       xyz[:, :, 2].reshape(-1), fps_idx.reshape(-1), table)
    out = _mlp(G, W0, b0, g0, be0, W1, b1, g1, be1, W2, b2, g2, be2)
    new_xyz = nxz.reshape(B, S, 3)
    new_points = out.reshape(B, S, 256)
    return new_xyz, new_points


# speedup vs baseline: 17.5689x; 1.0946x over previous
import functools

import jax
import jax.numpy as jnp
from jax import lax
from jax.experimental import pallas as pl
from jax.experimental.pallas import tpu as pltpu
from jax.experimental.pallas import tpu_sc as plsc

B = 8
N = 4096
S = 512
K = 32
CP = 125
CIN = 128
RADIUS2 = 0.2 ** 2
EPS = 1e-5
ROWS = B * S * K
BLK = 2048


def _tree_reduce(x, op):
    t = x
    w = N // 2
    while w >= 128:
        t = op(t[:, :w], t[:, w:2 * w])
        w //= 2
    return t


def _fps_body(x_ref, y_ref, z_ref, out_ref):
    X = x_ref[...]
    Y = y_ref[...]
    Z = z_ref[...]
    n_iota = lax.broadcasted_iota(jnp.int32, (B, N), 1)

    def red(x, op):
        t = _tree_reduce(x, op)
        if op is jnp.add:
            return jnp.sum(t, axis=1, keepdims=True)
        return jnp.max(t, axis=1, keepdims=True)

    def step(i, carry):
        dist, far = carry
        out_ref[i] = far
        onehot = n_iota == far
        cx = red(jnp.where(onehot, X, 0.0), jnp.add)
        cy = red(jnp.where(onehot, Y, 0.0), jnp.add)
        cz = red(jnp.where(onehot, Z, 0.0), jnp.add)
        d = (X - cx) ** 2 + (Y - cy) ** 2 + (Z - cz) ** 2
        dist = jnp.minimum(dist, d)
        m = red(dist, jnp.maximum)
        far = red(jnp.where(dist == m, n_iota, -1), jnp.maximum)
        return dist, far

    init = (jnp.full((B, N), 1e10, jnp.float32),
            jnp.zeros((B, 1), jnp.int32))
    lax.fori_loop(0, S, step, init)


def _fps(xyz):
    x = xyz[:, :, 0]
    y = xyz[:, :, 1]
    z = xyz[:, :, 2]
    out = pl.pallas_call(
        _fps_body,
        out_shape=jax.ShapeDtypeStruct((S, B, 1), jnp.int32),
    )(x, y, z)
    return out[:, :, 0].T


def _bf16_round(v):
    bits = plsc.bitcast(v, jnp.int32)
    lsb = jnp.bitwise_and(lax.shift_right_logical(bits, 16), 1)
    r = jnp.bitwise_and(bits + 0x7FFF + lsb, jnp.int32(-65536))
    return plsc.bitcast(r, jnp.float32)


def _sc_group(xf, yf, zf, fps_flat, table):
    mesh = plsc.VectorSubcoreMesh(core_axis_name="c", subcore_axis_name="s",
                                  num_cores=2, num_subcores=16)

    @functools.partial(
        pl.kernel,
        out_type=(jax.ShapeDtypeStruct((ROWS, CIN), jnp.float32),
                  jax.ShapeDtypeStruct((B * S * 3,), jnp.float32)),
        mesh=mesh,
        compiler_params=pltpu.CompilerParams(needs_layout_passes=False),
        scratch_types=[
            pltpu.VMEM((N,), jnp.float32),
            pltpu.VMEM((N,), jnp.float32),
            pltpu.VMEM((N,), jnp.float32),
            pltpu.VMEM((N,), jnp.float32),
            pltpu.VMEM((N,), jnp.float32),
            pltpu.VMEM((N,), jnp.float32),
            pltpu.VMEM((N,), jnp.float32),
            pltpu.VMEM((512,), jnp.int32),
            pltpu.VMEM((512,), jnp.float32),
            pltpu.VMEM((512,), jnp.float32),
            pltpu.VMEM((512,), jnp.float32),
            pltpu.VMEM((384,), jnp.float32),
            pltpu.VMEM((160,), jnp.int32),
            pltpu.VMEM((128,), jnp.int32),
            pltpu.VMEM((128,), jnp.int32),
            pltpu.VMEM((128,), jnp.int32),
            pltpu.VMEM((128,), jnp.int32),
            pltpu.VMEM((128, CIN), jnp.float32),
            pltpu.VMEM((128, CIN), jnp.float32),
            pltpu.VMEM((128, CIN), jnp.float32),
            pltpu.VMEM((128, CIN), jnp.float32),
            pltpu.SemaphoreType.DMA,
            pltpu.SemaphoreType.DMA,
            pltpu.SemaphoreType.DMA,
            pltpu.SemaphoreType.DMA,
        ],
    )
    def grouping(x_hbm, y_hbm, z_hbm, fps_hbm, t_hbm, g_hbm, nxz_hbm,
                 xv, yv, zv, xr, yr, zr, p2v, sidx, cxb, cyb, czb, nxs, idxb,
                 gl0A, gl0B, gl1A, gl1B, rb0A, rb0B, rb1A, rb1B,
                 gsem0, gsem1, osem0, osem1):
        cid = lax.axis_index("c")
        sid = lax.axis_index("s")
        w = sid * 2 + cid
        b = w // 4
        q = w % 4
        s0 = q * 128
        lane = lax.iota(jnp.int32, 16)

        pltpu.sync_copy(x_hbm.at[pl.ds(b * N, N)], xv)
        pltpu.sync_copy(y_hbm.at[pl.ds(b * N, N)], yv)
        pltpu.sync_copy(z_hbm.at[pl.ds(b * N, N)], zv)
        pltpu.sync_copy(fps_hbm.at[pl.ds(b * S, S)], sidx)

        def prep(t, _):
            bse = t * 16
            xs = xv[pl.ds(bse, 16)]
            ys = yv[pl.ds(bse, 16)]
            zs = zv[pl.ds(bse, 16)]
            xr[pl.ds(bse, 16)] = _bf16_round(xs)
            yr[pl.ds(bse, 16)] = _bf16_round(ys)
            zr[pl.ds(bse, 16)] = _bf16_round(zs)
            p2v[pl.ds(bse, 16)] = (xs * xs + ys * ys) + zs * zs
            return 0

        lax.fori_loop(0, N // 16, prep, 0)

        for j in range(32):
            iv = sidx[pl.ds(j * 16, 16)]
            cxb[pl.ds(j * 16, 16)] = plsc.load_gather(xv, [iv])
            cyb[pl.ds(j * 16, 16)] = plsc.load_gather(yv, [iv])
            czb[pl.ds(j * 16, 16)] = plsc.load_gather(zv, [iv])
        for j in range(8):
            offs = (j * 16 + lane) * 3
            plsc.store_scatter(nxs, [offs], cxb[pl.ds(s0 + j * 16, 16)])
            plsc.store_scatter(nxs, [offs + 1], cyb[pl.ds(s0 + j * 16, 16)])
            plsc.store_scatter(nxs, [offs + 2], czb[pl.ds(s0 + j * 16, 16)])
        pltpu.sync_copy(nxs, nxz_hbm.at[pl.ds((b * 512 + s0) * 3, 384)])

        def ball_query(sl, glist, slot):
            sv = jnp.full((16,), sl, jnp.int32)
            cxv = plsc.load_gather(cxb, [sv])
            cyv = plsc.load_gather(cyb, [sv])
            czv = plsc.load_gather(czb, [sv])
            c2 = (cxv * cxv + cyv * cyv) + czv * czv
            crx = _bf16_round(cxv)
            cry = _bf16_round(cyv)
            crz = _bf16_round(czv)
            idxb[pl.ds(0, 16)] = jnp.full((16,), N, jnp.int32)

            def cond(c):
                blk, cnt, _ = c
                return jnp.logical_and(cnt < 32, blk < N // 128)

            def body(c):
                blk, cnt, cntv = c
                base0 = blk * 128
                for j in range(8):
                    base = base0 + j * 16
                    xs = xr[pl.ds(base, 16)]
                    ys = yr[pl.ds(base, 16)]
                    zs = zr[pl.ds(base, 16)]
                    dot = (crx * xs + cry * ys) + crz * zs
                    d = ((-2.0 * dot) + c2) + p2v[pl.ds(base, 16)]
                    msk = jnp.logical_not(d > RADIUS2)
                    prefix = plsc.cumsum(jnp.where(msk, 1, 0))
                    plsc.store_scatter(idxb, [cntv + prefix - 1], base + lane,
                                       mask=msk)
                    cntv = cntv + plsc.all_reduce_population_count(msk)
                return blk + 1, jnp.max(cntv), cntv

            zerov = sv - sv
            _, cnt, _ = lax.while_loop(
                cond, body, (jnp.int32(0), jnp.int32(0), zerov))
            firstv = jnp.full((16,), jnp.min(idxb[pl.ds(0, 16)]), jnp.int32)
            for g in range(2):
                pos = g * 16 + lane
                cur = idxb[pl.ds(g * 16, 16)]
                sel = jnp.where(pos < cnt, cur, firstv)
                glist[pl.ds(slot * 32 + g * 16, 16)] = sel + b * N

        def bq_group(gi, glistA, glistB):
            sbase = (gi * 4 + q) * 8

            def ca(i, _):
                ball_query(sbase + i, glistA, i)
                return 0

            def cb(i, _):
                ball_query(sbase + 4 + i, glistB, i)
                return 0

            lax.fori_loop(0, 4, ca, 0)
            lax.fori_loop(0, 4, cb, 0)

        def process(gi, rbA, rbB, osem):
            sbase = (gi * 4 + q) * 8
            for half, rb in ((0, rbA), (1, rbB)):
                for j in range(8):
                    rid = j * 16 + lane
                    slv = sbase + half * 4 + lax.shift_right_logical(rid, 5)
                    for ci, cb in enumerate((cxb, cyb, czb)):
                        cv = plsc.load_gather(cb, [slv])
                        col = jnp.full((16,), ci, jnp.int32)
                        val = plsc.load_gather(rb, [rid, col]) - cv
                        plsc.store_scatter(rb, [rid, col], val)
            row0 = (b * 512 + sbase) * 32
            pltpu.async_copy(rbA, g_hbm.at[pl.ds(row0, 128)], osem)
            pltpu.async_copy(rbB, g_hbm.at[pl.ds(row0 + 128, 128)], osem)

        def fire_gather(glistA, glistB, rbA, rbB, sem):
            pltpu.async_copy(t_hbm.at[glistA], rbA, sem)
            pltpu.async_copy(t_hbm.at[glistB], rbB, sem)

        def drain2_gather(sem):
            pltpu.make_async_copy(t_hbm.at[pl.ds(0, 128)], rb0A, sem).wait()
            pltpu.make_async_copy(t_hbm.at[pl.ds(0, 128)], rb0B, sem).wait()

        def drain2_out(sem):
            pltpu.make_async_copy(rb0A, g_hbm.at[pl.ds(0, 128)], sem).wait()
            pltpu.make_async_copy(rb0B, g_hbm.at[pl.ds(0, 128)], sem).wait()

        bq_group(0, gl0A, gl0B)
        fire_gather(gl0A, gl0B, rb0A, rb0B, gsem0)

        def pipe(g2, _):
            base = g2 * 2

            bq_group(base + 1, gl1A, gl1B)

            @pl.when(g2 > 0)
            def _():
                drain2_out(osem1)
            fire_gather(gl1A, gl1B, rb1A, rb1B, gsem1)
            drain2_gather(gsem0)
            process(base, rb0A, rb0B, osem0)

            @pl.when(base + 2 < 16)
            def _():
                bq_group(base + 2, gl0A, gl0B)
                drain2_out(osem0)
                fire_gather(gl0A, gl0B, rb0A, rb0B, gsem0)
            drain2_gather(gsem1)
            process(base + 1, rb1A, rb1B, osem1)
            return 0

        lax.fori_loop(0, 8, pipe, 0)
        drain2_out(osem0)
        drain2_out(osem1)

    return grouping(xf, yf, zf, fps_flat, table)


def _stats(y):
    ssum = jnp.sum(y.reshape(BLK // 8, 8, y.shape[-1]), axis=0)
    ssq = jnp.sum((y * y).reshape(BLK // 8, 8, y.shape[-1]), axis=0)
    return jnp.concatenate([ssum, ssq], axis=0)


def _norm_from_stats(s_ref, g_ref, be_ref, y):
    s = s_ref[...]
    mean = jnp.sum(s[0:8], axis=0, keepdims=True) / ROWS
    ex2 = jnp.sum(s[8:16], axis=0, keepdims=True) / ROWS
    var = ex2 - mean * mean
    inv = lax.rsqrt(var + EPS)
    z = g_ref[...] * ((y - mean) * inv) + be_ref[...]
    return jnp.maximum(z, 0.0)


def _passA_body(g_ref, w_ref, b_ref, y_ref, s_ref):
    i = pl.program_id(0)
    y = jnp.dot(g_ref[...], w_ref[...],
                preferred_element_type=jnp.float32) + b_ref[...]
    y_ref[...] = y.astype(jnp.bfloat16)
    st = _stats(y)

    @pl.when(i == 0)
    def _():
        s_ref[...] = st

    @pl.when(i > 0)
    def _():
        s_ref[...] = s_ref[...] + st


def _passB_body(y0_ref, s0_ref, g0_ref, be0_ref, w_ref, b_ref, y_ref, s_ref):
    i = pl.program_id(0)
    x = _norm_from_stats(s0_ref, g0_ref, be0_ref,
                         y0_ref[...].astype(jnp.float32))
    y = jnp.dot(x, w_ref[...], preferred_element_type=jnp.float32) + b_ref[...]
    y_ref[...] = y.astype(jnp.bfloat16)
    st = _stats(y)

    @pl.when(i == 0)
    def _():
        s_ref[...] = st

    @pl.when(i > 0)
    def _():
        s_ref[...] = s_ref[...] + st


def _passC_body(y1_ref, s1_ref, g1_ref, be1_ref, w_ref, b_ref,
                mx_ref, mn_ref, s_ref):
    i = pl.program_id(0)
    x = _norm_from_stats(s1_ref, g1_ref, be1_ref,
                         y1_ref[...].astype(jnp.float32))
    y = jnp.dot(x, w_ref[...], preferred_element_type=jnp.float32) + b_ref[...]
    st = _stats(y)
    z = y.reshape(BLK // K, K, y.shape[-1])
    mx_ref[...] = jnp.max(z, axis=1)
    mn_ref[...] = jnp.min(z, axis=1)

    @pl.when(i == 0)
    def _():
        s_ref[...] = st

    @pl.when(i > 0)
    def _():
        s_ref[...] = s_ref[...] + st


def _passD_body(mx_ref, mn_ref, s2_ref, g2_ref, be2_ref, o_ref):
    g2 = g2_ref[...]
    sel = jnp.where(g2 >= 0.0, mx_ref[...], mn_ref[...])
    o_ref[...] = _norm_from_stats(s2_ref, g2_ref, be2_ref, sel)


def _mlp(G, W0, b0, g0, be0, W1, b1, g1, be1, W2, b2, g2, be2):
    f32 = jnp.float32
    grid = (ROWS // BLK,)
    row_spec = lambda c: pl.BlockSpec((BLK, c), lambda i: (i, 0))
    const_spec = lambda r, c: pl.BlockSpec((r, c), lambda i: (0, 0))

    Y0, S0 = pl.pallas_call(
        _passA_body,
        grid=grid,
        in_specs=[row_spec(CIN), const_spec(CIN, 128), const_spec(1, 128)],
        out_specs=[row_spec(128), const_spec(16, 128)],
        out_shape=[jax.ShapeDtypeStruct((ROWS, 128), jnp.bfloat16),
                   jax.ShapeDtypeStruct((16, 128), f32)],
    )(G, W0.T, b0[None, :])

    Y1, S1 = pl.pallas_call(
        _passB_body,
        grid=grid,
        in_specs=[row_spec(128), const_spec(16, 128), const_spec(1, 128),
                  const_spec(1, 128), const_spec(128, 128), const_spec(1, 128)],
        out_specs=[row_spec(128), const_spec(16, 128)],
        out_shape=[jax.ShapeDtypeStruct((ROWS, 128), jnp.bfloat16),
                   jax.ShapeDtypeStruct((16, 128), f32)],
    )(Y0, S0, g0[None, :], be0[None, :], W1.T, b1[None, :])

    MX, MN, S2 = pl.pallas_call(
        _passC_body,
        grid=grid,
        in_specs=[row_spec(128), const_spec(16, 128), const_spec(1, 128),
                  const_spec(1, 128), const_spec(128, 256), const_spec(1, 256)],
        out_specs=[pl.BlockSpec((BLK // K, 256), lambda i: (i, 0)),
                   pl.BlockSpec((BLK // K, 256), lambda i: (i, 0)),
                   const_spec(16, 256)],
        out_shape=[jax.ShapeDtypeStruct((ROWS // K, 256), f32),
                   jax.ShapeDtypeStruct((ROWS // K, 256), f32),
                   jax.ShapeDtypeStruct((16, 256), f32)],
    )(Y1, S1, g1[None, :], be1[None, :], W2.T, b2[None, :])

    out = pl.pallas_call(
        _passD_body,
        out_shape=jax.ShapeDtypeStruct((ROWS // K, 256), f32),
    )(MX, MN, S2, g2[None, :], be2[None, :])
    return out


def kernel(xyz, points, W0, b0, g0, be0, W1, b1, g1, be1, W2, b2, g2, be2):
    fps_idx = _fps(xyz)
    table = jnp.concatenate([xyz, points], axis=-1)
    table = table.reshape(B * N, CIN)
    G, nxz = _sc_group(xyz[:, :, 0].reshape(-1), xyz[:, :, 1].reshape(-1),
                       xyz[:, :, 2].reshape(-1), fps_idx.reshape(-1), table)
    out = _mlp(G, W0, b0, g0, be0, W1, b1, g1, be1, W2, b2, g2, be2)
    new_xyz = nxz.reshape(B, S, 3)
    new_points = out.reshape(B, S, 256)
    return new_xyz, new_points
